# baseline (speedup 1.0000x reference)
import functools

import jax
import jax.numpy as jnp
from jax import lax
from jax.experimental import pallas as pl
from jax.experimental.pallas import tpu as pltpu
from jax.experimental.pallas import tpu_sc as plsc

NP = 10240
RPS = NP // 16
TRASH = NP - 1
CH = 128
BLK = 256
F32 = jnp.float32


def _sc_mesh():
    return plsc.VectorSubcoreMesh(core_axis_name="c", subcore_axis_name="s")


NBUF = 2


@functools.lru_cache(maxsize=None)
def _segsum(nt, d, e):
    assert e % (32 * CH * NBUF) == 0
    n_w = e // (32 * CH)
    dd = d // 16

    @functools.partial(
        pl.kernel,
        mesh=_sc_mesh(),
        out_type=[
            jax.ShapeDtypeStruct((2 * NP, d), F32),
            jax.ShapeDtypeStruct((2 * NP,), F32),
        ],
        scratch_types=[
            pltpu.VMEM((CH,), jnp.int32),
            pltpu.VMEM((CH,), jnp.int32),
            pltpu.VMEM((CH,), jnp.int32),
            pltpu.VMEM((CH,), jnp.int32),
            pltpu.VMEM((CH, d), F32),
            pltpu.VMEM((CH, d), F32),
            pltpu.VMEM((CH,), F32),
            pltpu.VMEM_SHARED((NP, d), F32),
            pltpu.VMEM_SHARED((NP,), F32),
            pltpu.SemaphoreType.DMA,
            pltpu.SemaphoreType.DMA,
        ],
    )
    def k(table, srcl, dstl, out, cnt_out, src0, src1, dst0, dst1,
          rows0, rows1, ones_v, acc_sh, cnt_sh, gs0, gs1):
        srcs = (src0, src1)
        dsts = (dst0, dst1)
        rows = (rows0, rows1)
        gsem = (gs0, gs1)
        c = lax.axis_index("c")
        s = lax.axis_index("s")
        w = s * 2 + c

        def zero_body(i, carry):
            rows0[i // dd, pl.ds((i % dd) * 16, 16)] = jnp.zeros((16,), F32)
            return carry

        lax.fori_loop(0, CH * dd, zero_body, 0)
        base = s * RPS
        for j in range(RPS // CH):
            pltpu.sync_copy(rows0, acc_sh.at[pl.ds(base + j * CH, CH)])
            pltpu.sync_copy(rows0.at[0], cnt_sh.at[pl.ds(base + j * CH, CH)])
        for j in range(CH // 16):
            ones_v[pl.ds(j * 16, 16)] = jnp.ones((16,), F32)
        plsc.subcore_barrier()

        def load_and_fire(b, i):
            bb = (w + i * 32) * CH
            pltpu.sync_copy(srcl.at[pl.ds(bb, CH)], srcs[b])
            pltpu.sync_copy(dstl.at[pl.ds(bb, CH)], dsts[b])
            pltpu.async_copy(table.at[srcs[b]], rows[b], gsem[b])

        def drain_and_scatter(b):
            pltpu.make_async_copy(table.at[srcs[b]], rows[b], gsem[b]).wait()
            pltpu.sync_copy(rows[b], acc_sh.at[dsts[b]], add=True)
            pltpu.sync_copy(ones_v, cnt_sh.at[dsts[b]], add=True)

        for b in range(NBUF):
            load_and_fire(b, b)

        def body(g, carry):
            for b in range(NBUF):
                i = g * NBUF + b
                drain_and_scatter(b)
                load_and_fire(b, i + NBUF)
            return carry

        lax.fori_loop(0, (n_w - NBUF) // NBUF, body, 0)
        for b in range(NBUF):
            drain_and_scatter(b)
        plsc.subcore_barrier()
        ob = c * NP + base
        pltpu.sync_copy(acc_sh.at[pl.ds(base, RPS)], out.at[pl.ds(ob, RPS)])
        pltpu.sync_copy(cnt_sh.at[pl.ds(base, RPS)],
                        cnt_out.at[pl.ds(ob, RPS)])

    return k


def _trash(num):
    return 10000 + (jnp.arange(num, dtype=jnp.int32) % (NP - 10000))


def _pad_edges(src, dst, e):
    ep = -(-e // 8192) * 8192
    if ep == e:
        return src, dst, e
    pad = ep - e
    src_p = jnp.concatenate([src, jnp.zeros((pad,), jnp.int32)])
    dst_p = jnp.concatenate([dst, _trash(pad)])
    return src_p, dst_p, ep


@functools.lru_cache(maxsize=None)
def _gather(nt, d):
    bpw = 1024 // 32

    @functools.partial(
        pl.kernel,
        mesh=_sc_mesh(),
        out_type=jax.ShapeDtypeStruct((1024, d), F32),
        scratch_types=[
            pltpu.VMEM((bpw,), jnp.int32),
            pltpu.VMEM((bpw, d), F32),
            pltpu.SemaphoreType.DMA,
        ],
    )
    def k(table, idx, out, idx_v, rows_v, sem):
        w = lax.axis_index("s") * 2 + lax.axis_index("c")
        base = w * bpw
        pltpu.sync_copy(idx.at[pl.ds(base, bpw)], idx_v)
        pltpu.async_copy(table.at[idx_v], rows_v, sem).wait()
        pltpu.sync_copy(rows_v, out.at[pl.ds(base, bpw)])

    return k


def _mm(a, w):
    return jnp.dot(a, w, preferred_element_type=F32)


def _gen_body(aggA, aggB, cntA, cntB, xb, wgl, bgl, wgr, wd1, bd1, wd2, bd2,
              gen_o):
    cnt = cntA[...] + cntB[...]
    mean0 = (aggA[...] + aggB[...]) / jnp.maximum(cnt, 1.0)
    h = jnp.maximum(_mm(mean0, wgl[...]) + bgl[...] + _mm(xb[...], wgr[...]),
                    0.0)
    t = jnp.maximum(_mm(h, wd1[...]) + bd1[...], 0.0)
    gen_o[...] = _mm(t, wd2[...]) + bd2[...]


def _conv1_body(aggA, aggB, e1A, e1B, cntA, cntB, kA, kB, xb, wl1, bl1, wr1,
                h1lo_o, h1hi_o, den_o):
    den = jnp.maximum(cntA[...] + cntB[...] + kA[...] + kB[...], 1.0)
    den_r = 1.0 / den
    mean1 = (aggA[...] + aggB[...] + e1A[...] + e1B[...]) * den_r
    h1 = jnp.maximum(_mm(mean1, wl1[...]) + bl1[...] + _mm(xb[...], wr1[...]),
                     0.0)
    h1lo_o[...] = h1[:, :128]
    h1hi_o[...] = h1[:, 128:]
    den_o[...] = den_r


def _new1_body(xm, gm, wl1, bl1, wr1, lo_o, hi_o):
    h1n = jnp.maximum(_mm(xm[...], wl1[...]) + bl1[...] +
                      _mm(gm[...], wr1[...]), 0.0)
    lo_o[...] = h1n[:, :128]
    hi_o[...] = h1n[:, 128:]


def _conv2_body(aloA, aloB, ahiA, ahiB, eloA, eloB, ehiA, ehiB, den, h1lo,
                h1hi, wl2, bl2, wr2, wp, bp, out_o):
    d = den[...]
    mlo = (aloA[...] + aloB[...] + eloA[...] + eloB[...]) * d
    mhi = (ahiA[...] + ahiB[...] + ehiA[...] + ehiB[...]) * d
    wl2v = wl2[...]
    wr2v = wr2[...]
    h2 = jnp.maximum(
        _mm(mlo, wl2v[:128]) + _mm(mhi, wl2v[128:]) + bl2[...] +
        _mm(h1lo[...], wr2v[:128]) + _mm(h1hi[...], wr2v[128:]), 0.0)
    out_o[...] = _mm(h2, wp[...]) + bp[...]


def _new2_body(h1mlo, h1mhi, h1nlo, h1nhi, wl2, bl2, wr2, wp, bp, out_o):
    wl2v = wl2[...]
    wr2v = wr2[...]
    h2n = jnp.maximum(
        _mm(h1mlo[...], wl2v[:128]) + _mm(h1mhi[...], wl2v[128:]) + bl2[...] +
        _mm(h1nlo[...], wr2v[:128]) + _mm(h1nhi[...], wr2v[128:]), 0.0)
    out_o[...] = _mm(h2n, wp[...]) + bp[...]


def _row_spec(w, two_part):
    nb = NP // BLK
    if two_part == 0:
        return pl.BlockSpec((BLK, w), lambda i: (i, 0))
    return pl.BlockSpec((BLK, w), lambda i, nb=nb: (i + nb, 0))


def _full_spec(shape):
    nd = len(shape)
    return pl.BlockSpec(shape, lambda i: (0,) * nd)


def kernel(x, edge_index, missing_indices, Wl1, bl1, Wr1, Wl2, bl2, Wr2,
           Wp, bp, Wgl, bgl, Wgr, Wd1, bd1, Wd2, bd2):
    n, dx = x.shape
    e = edge_index.shape[1]
    m = missing_indices.shape[0]
    src = edge_index[0].astype(jnp.int32)
    dst = edge_index[1].astype(jnp.int32)
    midx = missing_indices.astype(jnp.int32)
    mp = 1024
    x_pad = jnp.pad(x, ((0, NP - n), (0, 0)))
    src_p, dst_p, ep = _pad_edges(src, dst, e)
    src_m = jnp.concatenate([midx, jnp.zeros((mp - m,), jnp.int32)])
    src_m8 = jnp.concatenate([midx, jnp.zeros((8192 - m,), jnp.int32)])
    dst_m8 = jnp.concatenate([midx, _trash(8192 - m)])
    ar8 = jnp.concatenate([jnp.arange(mp, dtype=jnp.int32),
                           jnp.zeros((8192 - mp,), jnp.int32)])

    bgl_r = bgl.reshape(1, -1)
    bd1_r = bd1.reshape(1, -1)
    bd2_r = bd2.reshape(1, -1)
    bl1_r = bl1.reshape(1, -1)
    bl2_r = bl2.reshape(1, -1)
    bp_r = bp.reshape(1, -1)

    agg, cnt = _segsum(NP, 128, ep)(x_pad, src_p, dst_p)
    cnt2 = cnt.reshape(2 * NP, 1)

    nb = NP // BLK
    gen = pl.pallas_call(
        _gen_body,
        grid=(nb,),
        in_specs=[
            _row_spec(128, 0), _row_spec(128, 1),
            _row_spec(1, 0), _row_spec(1, 1),
            _row_spec(128, 0),
            _full_spec((128, 256)), _full_spec((1, 256)),
            _full_spec((128, 256)),
            _full_spec((256, 256)), _full_spec((1, 256)),
            _full_spec((256, 128)), _full_spec((1, 128)),
        ],
        out_specs=_row_spec(128, 0),
        out_shape=jax.ShapeDtypeStruct((NP, 128), F32),
    )(agg, agg, cnt2, cnt2, x_pad, Wgl, bgl_r, Wgr, Wd1, bd1_r, Wd2, bd2_r)

    xm = _gather(NP, 128)(x_pad, src_m)
    gm = _gather(NP, 128)(gen, src_m)
    e1, kcnt = _segsum(NP, 128, 8192)(gen, src_m8, dst_m8)
    k2 = kcnt.reshape(2 * NP, 1)

    h1lo, h1hi, den_r = pl.pallas_call(
        _conv1_body,
        grid=(nb,),
        in_specs=[
            _row_spec(128, 0), _row_spec(128, 1),
            _row_spec(128, 0), _row_spec(128, 1),
            _row_spec(1, 0), _row_spec(1, 1),
            _row_spec(1, 0), _row_spec(1, 1),
            _row_spec(128, 0),
            _full_spec((128, 256)), _full_spec((1, 256)),
            _full_spec((128, 256)),
        ],
        out_specs=[_row_spec(128, 0), _row_spec(128, 0), _row_spec(1, 0)],
        out_shape=[
            jax.ShapeDtypeStruct((NP, 128), F32),
            jax.ShapeDtypeStruct((NP, 128), F32),
            jax.ShapeDtypeStruct((NP, 1), F32),
        ],
    )(agg, agg, e1, e1, cnt2, cnt2, k2, k2, x_pad, Wl1, bl1_r, Wr1)

    h1nlo, h1nhi = pl.pallas_call(
        _new1_body,
        grid=(mp // BLK,),
        in_specs=[
            _row_spec(128, 0), _row_spec(128, 0),
            _full_spec((128, 256)), _full_spec((1, 256)),
            _full_spec((128, 256)),
        ],
        out_specs=[_row_spec(128, 0), _row_spec(128, 0)],
        out_shape=[
            jax.ShapeDtypeStruct((mp, 128), F32),
            jax.ShapeDtypeStruct((mp, 128), F32),
        ],
    )(xm, gm, Wl1, bl1_r, Wr1)

    a2lo, _ = _segsum(NP, 128, ep)(h1lo, src_p, dst_p)
    a2hi, _ = _segsum(NP, 128, ep)(h1hi, src_p, dst_p)
    e2lo, _ = _segsum(1024, 128, 8192)(h1nlo, ar8, dst_m8)
    e2hi, _ = _segsum(1024, 128, 8192)(h1nhi, ar8, dst_m8)
    h1mlo = _gather(NP, 128)(h1lo, src_m)
    h1mhi = _gather(NP, 128)(h1hi, src_m)

    out_main = pl.pallas_call(
        _conv2_body,
        grid=(nb,),
        in_specs=[
            _row_spec(128, 0), _row_spec(128, 1),
            _row_spec(128, 0), _row_spec(128, 1),
            _row_spec(128, 0), _row_spec(128, 1),
            _row_spec(128, 0), _row_spec(128, 1),
            _row_spec(1, 0),
            _row_spec(128, 0), _row_spec(128, 0),
            _full_spec((256, 256)), _full_spec((1, 256)),
            _full_spec((256, 256)),
            _full_spec((256, 64)), _full_spec((1, 64)),
        ],
        out_specs=_row_spec(64, 0),
        out_shape=jax.ShapeDtypeStruct((NP, 64), F32),
    )(a2lo, a2lo, a2hi, a2hi, e2lo, e2lo, e2hi, e2hi, den_r, h1lo, h1hi,
      Wl2, bl2_r, Wr2, Wp, bp_r)

    out_new = pl.pallas_call(
        _new2_body,
        grid=(mp // BLK,),
        in_specs=[
            _row_spec(128, 0), _row_spec(128, 0),
            _row_spec(128, 0), _row_spec(128, 0),
            _full_spec((256, 256)), _full_spec((1, 256)),
            _full_spec((256, 256)),
            _full_spec((256, 64)), _full_spec((1, 64)),
        ],
        out_specs=_row_spec(64, 0),
        out_shape=jax.ShapeDtypeStruct((mp, 64), F32),
    )(h1mlo, h1mhi, h1nlo, h1nhi, Wl2, bl2_r, Wr2, Wp, bp_r)

    return jnp.concatenate([out_main[:n], out_new[:m]], axis=0)


# speedup vs baseline: 3.2919x; 3.2919x over previous
import functools

import jax
import jax.numpy as jnp
from jax import lax
from jax.experimental import pallas as pl
from jax.experimental.pallas import tpu as pltpu
from jax.experimental.pallas import tpu_sc as plsc

NP = 10240
RPS = NP // 16
TRASH = NP - 1
CH = 128
BLK = 256
F32 = jnp.float32


def _sc_mesh():
    return plsc.VectorSubcoreMesh(core_axis_name="c", subcore_axis_name="s")


NBUF = 2


@functools.lru_cache(maxsize=None)
def _segsum(nt, d, e):
    assert e % (32 * CH * NBUF) == 0
    n_w = e // (32 * CH)
    dd = d // 16

    @functools.partial(
        pl.kernel,
        mesh=_sc_mesh(),
        out_type=[
            jax.ShapeDtypeStruct((2 * NP, d), F32),
            jax.ShapeDtypeStruct((2 * NP,), F32),
        ],
        scratch_types=[
            pltpu.VMEM((CH,), jnp.int32),
            pltpu.VMEM((CH,), jnp.int32),
            pltpu.VMEM((CH,), jnp.int32),
            pltpu.VMEM((CH,), jnp.int32),
            pltpu.VMEM((CH, d), F32),
            pltpu.VMEM((CH, d), F32),
            pltpu.VMEM((CH,), F32),
            pltpu.VMEM_SHARED((NP, d), F32),
            pltpu.VMEM_SHARED((NP,), F32),
            pltpu.SemaphoreType.DMA,
            pltpu.SemaphoreType.DMA,
        ],
    )
    def k(table, srcl, dstl, out, cnt_out, src0, src1, dst0, dst1,
          rows0, rows1, ones_v, acc_sh, cnt_sh, gs0, gs1):
        srcs = (src0, src1)
        dsts = (dst0, dst1)
        rows = (rows0, rows1)
        gsem = (gs0, gs1)
        c = lax.axis_index("c")
        s = lax.axis_index("s")
        w = s * 2 + c

        def zero_body(i, carry):
            rows0[i // dd, pl.ds((i % dd) * 16, 16)] = jnp.zeros((16,), F32)
            return carry

        lax.fori_loop(0, CH * dd, zero_body, 0)
        base = s * RPS
        for j in range(RPS // CH):
            pltpu.sync_copy(rows0, acc_sh.at[pl.ds(base + j * CH, CH)])
            pltpu.sync_copy(rows0.at[0], cnt_sh.at[pl.ds(base + j * CH, CH)])
        for j in range(CH // 16):
            ones_v[pl.ds(j * 16, 16)] = jnp.ones((16,), F32)
        plsc.subcore_barrier()

        def load_and_fire(b, i):
            bb = (w + i * 32) * CH
            pltpu.sync_copy(srcl.at[pl.ds(bb, CH)], srcs[b])
            pltpu.sync_copy(dstl.at[pl.ds(bb, CH)], dsts[b])
            pltpu.async_copy(table.at[srcs[b]], rows[b], gsem[b])

        def drain_and_scatter(b):
            pltpu.make_async_copy(table.at[srcs[b]], rows[b], gsem[b]).wait()
            pltpu.sync_copy(rows[b], acc_sh.at[dsts[b]], add=True)
            pltpu.sync_copy(ones_v, cnt_sh.at[dsts[b]], add=True)

        for b in range(NBUF):
            load_and_fire(b, b)

        def body(g, carry):
            for b in range(NBUF):
                i = g * NBUF + b
                drain_and_scatter(b)
                load_and_fire(b, i + NBUF)
            return carry

        lax.fori_loop(0, (n_w - NBUF) // NBUF, body, 0)
        for b in range(NBUF):
            drain_and_scatter(b)
        plsc.subcore_barrier()
        ob = c * NP + base
        pltpu.sync_copy(acc_sh.at[pl.ds(base, RPS)], out.at[pl.ds(ob, RPS)])
        pltpu.sync_copy(cnt_sh.at[pl.ds(base, RPS)],
                        cnt_out.at[pl.ds(ob, RPS)])

    return k


def _trash(num):
    return 10000 + (jnp.arange(num, dtype=jnp.int32) % (NP - 10000))


def _spread_src(num, nt):
    return jnp.arange(num, dtype=jnp.int32) % nt


def _pad_edges(src, dst, e, nt):
    ep = -(-e // 8192) * 8192
    if ep == e:
        return src, dst, e
    pad = ep - e
    src_p = jnp.concatenate([src, _spread_src(pad, nt)])
    dst_p = jnp.concatenate([dst, _trash(pad)])
    return src_p, dst_p, ep


@functools.lru_cache(maxsize=None)
def _gather(nt, d):
    bpw = 1024 // 32

    @functools.partial(
        pl.kernel,
        mesh=_sc_mesh(),
        out_type=jax.ShapeDtypeStruct((1024, d), F32),
        scratch_types=[
            pltpu.VMEM((bpw,), jnp.int32),
            pltpu.VMEM((bpw, d), F32),
            pltpu.SemaphoreType.DMA,
        ],
    )
    def k(table, idx, out, idx_v, rows_v, sem):
        w = lax.axis_index("s") * 2 + lax.axis_index("c")
        base = w * bpw
        pltpu.sync_copy(idx.at[pl.ds(base, bpw)], idx_v)
        pltpu.async_copy(table.at[idx_v], rows_v, sem).wait()
        pltpu.sync_copy(rows_v, out.at[pl.ds(base, bpw)])

    return k


def _mm(a, w):
    return jnp.dot(a, w, preferred_element_type=F32)


def _gen_body(aggA, aggB, cntA, cntB, xb, wgl, bgl, wgr, wd1, bd1, wd2, bd2,
              gen_o):
    cnt = cntA[...] + cntB[...]
    mean0 = (aggA[...] + aggB[...]) / jnp.maximum(cnt, 1.0)
    h = jnp.maximum(_mm(mean0, wgl[...]) + bgl[...] + _mm(xb[...], wgr[...]),
                    0.0)
    t = jnp.maximum(_mm(h, wd1[...]) + bd1[...], 0.0)
    gen_o[...] = _mm(t, wd2[...]) + bd2[...]


def _conv1_body(aggA, aggB, e1A, e1B, cntA, cntB, kA, kB, xb, wl1, bl1, wr1,
                h1lo_o, h1hi_o, den_o):
    den = jnp.maximum(cntA[...] + cntB[...] + kA[...] + kB[...], 1.0)
    den_r = 1.0 / den
    mean1 = (aggA[...] + aggB[...] + e1A[...] + e1B[...]) * den_r
    h1 = jnp.maximum(_mm(mean1, wl1[...]) + bl1[...] + _mm(xb[...], wr1[...]),
                     0.0)
    h1lo_o[...] = h1[:, :128]
    h1hi_o[...] = h1[:, 128:]
    den_o[...] = den_r


def _new1_body(xm, gm, wl1, bl1, wr1, lo_o, hi_o):
    h1n = jnp.maximum(_mm(xm[...], wl1[...]) + bl1[...] +
                      _mm(gm[...], wr1[...]), 0.0)
    lo_o[...] = h1n[:, :128]
    hi_o[...] = h1n[:, 128:]


def _conv2_body(aloA, aloB, ahiA, ahiB, eloA, eloB, ehiA, ehiB, den, h1lo,
                h1hi, wl2, bl2, wr2, wp, bp, out_o):
    d = den[...]
    mlo = (aloA[...] + aloB[...] + eloA[...] + eloB[...]) * d
    mhi = (ahiA[...] + ahiB[...] + ehiA[...] + ehiB[...]) * d
    wl2v = wl2[...]
    wr2v = wr2[...]
    h2 = jnp.maximum(
        _mm(mlo, wl2v[:128]) + _mm(mhi, wl2v[128:]) + bl2[...] +
        _mm(h1lo[...], wr2v[:128]) + _mm(h1hi[...], wr2v[128:]), 0.0)
    out_o[...] = _mm(h2, wp[...]) + bp[...]


def _new2_body(h1mlo, h1mhi, h1nlo, h1nhi, wl2, bl2, wr2, wp, bp, out_o):
    wl2v = wl2[...]
    wr2v = wr2[...]
    h2n = jnp.maximum(
        _mm(h1mlo[...], wl2v[:128]) + _mm(h1mhi[...], wl2v[128:]) + bl2[...] +
        _mm(h1nlo[...], wr2v[:128]) + _mm(h1nhi[...], wr2v[128:]), 0.0)
    out_o[...] = _mm(h2n, wp[...]) + bp[...]


def _row_spec(w, two_part):
    nb = NP // BLK
    if two_part == 0:
        return pl.BlockSpec((BLK, w), lambda i: (i, 0))
    return pl.BlockSpec((BLK, w), lambda i, nb=nb: (i + nb, 0))


def _full_spec(shape):
    nd = len(shape)
    return pl.BlockSpec(shape, lambda i: (0,) * nd)


def kernel(x, edge_index, missing_indices, Wl1, bl1, Wr1, Wl2, bl2, Wr2,
           Wp, bp, Wgl, bgl, Wgr, Wd1, bd1, Wd2, bd2):
    n, dx = x.shape
    e = edge_index.shape[1]
    m = missing_indices.shape[0]
    src = edge_index[0].astype(jnp.int32)
    dst = edge_index[1].astype(jnp.int32)
    midx = missing_indices.astype(jnp.int32)
    mp = 1024
    x_pad = jnp.pad(x, ((0, NP - n), (0, 0)))
    src_p, dst_p, ep = _pad_edges(src, dst, e, n)
    src_m = jnp.concatenate([midx, jnp.zeros((mp - m,), jnp.int32)])
    src_m8 = jnp.concatenate([midx, _spread_src(8192 - m, n)])
    dst_m8 = jnp.concatenate([midx, _trash(8192 - m)])
    ar8 = jnp.concatenate([jnp.arange(mp, dtype=jnp.int32),
                           _spread_src(8192 - mp, mp)])

    bgl_r = bgl.reshape(1, -1)
    bd1_r = bd1.reshape(1, -1)
    bd2_r = bd2.reshape(1, -1)
    bl1_r = bl1.reshape(1, -1)
    bl2_r = bl2.reshape(1, -1)
    bp_r = bp.reshape(1, -1)

    agg, cnt = _segsum(NP, 128, ep)(x_pad, src_p, dst_p)
    cnt2 = cnt.reshape(2 * NP, 1)

    nb = NP // BLK
    gen = pl.pallas_call(
        _gen_body,
        grid=(nb,),
        in_specs=[
            _row_spec(128, 0), _row_spec(128, 1),
            _row_spec(1, 0), _row_spec(1, 1),
            _row_spec(128, 0),
            _full_spec((128, 256)), _full_spec((1, 256)),
            _full_spec((128, 256)),
            _full_spec((256, 256)), _full_spec((1, 256)),
            _full_spec((256, 128)), _full_spec((1, 128)),
        ],
        out_specs=_row_spec(128, 0),
        out_shape=jax.ShapeDtypeStruct((NP, 128), F32),
    )(agg, agg, cnt2, cnt2, x_pad, Wgl, bgl_r, Wgr, Wd1, bd1_r, Wd2, bd2_r)

    xm = _gather(NP, 128)(x_pad, src_m)
    gm = _gather(NP, 128)(gen, src_m)
    e1, kcnt = _segsum(NP, 128, 8192)(gen, src_m8, dst_m8)
    k2 = kcnt.reshape(2 * NP, 1)

    h1lo, h1hi, den_r = pl.pallas_call(
        _conv1_body,
        grid=(nb,),
        in_specs=[
            _row_spec(128, 0), _row_spec(128, 1),
            _row_spec(128, 0), _row_spec(128, 1),
            _row_spec(1, 0), _row_spec(1, 1),
            _row_spec(1, 0), _row_spec(1, 1),
            _row_spec(128, 0),
            _full_spec((128, 256)), _full_spec((1, 256)),
            _full_spec((128, 256)),
        ],
        out_specs=[_row_spec(128, 0), _row_spec(128, 0), _row_spec(1, 0)],
        out_shape=[
            jax.ShapeDtypeStruct((NP, 128), F32),
            jax.ShapeDtypeStruct((NP, 128), F32),
            jax.ShapeDtypeStruct((NP, 1), F32),
        ],
    )(agg, agg, e1, e1, cnt2, cnt2, k2, k2, x_pad, Wl1, bl1_r, Wr1)

    h1nlo, h1nhi = pl.pallas_call(
        _new1_body,
        grid=(mp // BLK,),
        in_specs=[
            _row_spec(128, 0), _row_spec(128, 0),
            _full_spec((128, 256)), _full_spec((1, 256)),
            _full_spec((128, 256)),
        ],
        out_specs=[_row_spec(128, 0), _row_spec(128, 0)],
        out_shape=[
            jax.ShapeDtypeStruct((mp, 128), F32),
            jax.ShapeDtypeStruct((mp, 128), F32),
        ],
    )(xm, gm, Wl1, bl1_r, Wr1)

    a2lo, _ = _segsum(NP, 128, ep)(h1lo, src_p, dst_p)
    a2hi, _ = _segsum(NP, 128, ep)(h1hi, src_p, dst_p)
    e2lo, _ = _segsum(1024, 128, 8192)(h1nlo, ar8, dst_m8)
    e2hi, _ = _segsum(1024, 128, 8192)(h1nhi, ar8, dst_m8)
    h1mlo = _gather(NP, 128)(h1lo, src_m)
    h1mhi = _gather(NP, 128)(h1hi, src_m)

    out_main = pl.pallas_call(
        _conv2_body,
        grid=(nb,),
        in_specs=[
            _row_spec(128, 0), _row_spec(128, 1),
            _row_spec(128, 0), _row_spec(128, 1),
            _row_spec(128, 0), _row_spec(128, 1),
            _row_spec(128, 0), _row_spec(128, 1),
            _row_spec(1, 0),
            _row_spec(128, 0), _row_spec(128, 0),
            _full_spec((256, 256)), _full_spec((1, 256)),
            _full_spec((256, 256)),
            _full_spec((256, 64)), _full_spec((1, 64)),
        ],
        out_specs=_row_spec(64, 0),
        out_shape=jax.ShapeDtypeStruct((NP, 64), F32),
    )(a2lo, a2lo, a2hi, a2hi, e2lo, e2lo, e2hi, e2hi, den_r, h1lo, h1hi,
      Wl2, bl2_r, Wr2, Wp, bp_r)

    out_new = pl.pallas_call(
        _new2_body,
        grid=(mp // BLK,),
        in_specs=[
            _row_spec(128, 0), _row_spec(128, 0),
            _row_spec(128, 0), _row_spec(128, 0),
            _full_spec((256, 256)), _full_spec((1, 256)),
            _full_spec((256, 256)),
            _full_spec((256, 64)), _full_spec((1, 64)),
        ],
        out_specs=_row_spec(64, 0),
        out_shape=jax.ShapeDtypeStruct((mp, 64), F32),
    )(h1mlo, h1mhi, h1nlo, h1nhi, Wl2, bl2_r, Wr2, Wp, bp_r)

    return jnp.concatenate([out_main[:n], out_new[:m]], axis=0)


# speedup vs baseline: 3.3439x; 1.0158x over previous
import functools

import jax
import jax.numpy as jnp
from jax import lax
from jax.experimental import pallas as pl
from jax.experimental.pallas import tpu as pltpu
from jax.experimental.pallas import tpu_sc as plsc

NP = 10240
RPS = NP // 16
TRASH = NP - 1
CH = 128
BLK = 256
F32 = jnp.float32


def _sc_mesh():
    return plsc.VectorSubcoreMesh(core_axis_name="c", subcore_axis_name="s")


NBUF = 2


def _zero_rows(rows0, dd):
    def zero_body(i, carry):
        rows0[i // dd, pl.ds((i % dd) * 16, 16)] = jnp.zeros((16,), F32)
        return carry

    lax.fori_loop(0, CH * dd, zero_body, 0)


def _zero_acc(rows0, base, acc_sh, cnt_sh):
    for j in range(RPS // CH):
        pltpu.sync_copy(rows0, acc_sh.at[pl.ds(base + j * CH, CH)])
        if cnt_sh is not None:
            pltpu.sync_copy(rows0.at[0], cnt_sh.at[pl.ds(base + j * CH, CH)])


def _edge_loop(n_w, w, table, srcl, dstl, srcs, dsts, rows, gsem, ones_v,
               acc_sh, cnt_sh):
    def load_and_fire(b, i):
        bb = (w + i * 32) * CH
        pltpu.sync_copy(srcl.at[pl.ds(bb, CH)], srcs[b])
        pltpu.sync_copy(dstl.at[pl.ds(bb, CH)], dsts[b])
        pltpu.async_copy(table.at[srcs[b]], rows[b], gsem[b])

    def drain_and_scatter(b):
        pltpu.make_async_copy(table.at[srcs[b]], rows[b], gsem[b]).wait()
        pltpu.sync_copy(rows[b], acc_sh.at[dsts[b]], add=True)
        if cnt_sh is not None:
            pltpu.sync_copy(ones_v, cnt_sh.at[dsts[b]], add=True)

    for b in range(NBUF):
        load_and_fire(b, b)

    def body(g, carry):
        for b in range(NBUF):
            i = g * NBUF + b
            drain_and_scatter(b)
            load_and_fire(b, i + NBUF)
        return carry

    lax.fori_loop(0, (n_w - NBUF) // NBUF, body, 0)
    for b in range(NBUF):
        drain_and_scatter(b)


def _gather32(table, idxg, out_o, w, idx_v, grow_v, sem):
    base = w * 32
    pltpu.sync_copy(idxg.at[pl.ds(base, 32)], idx_v)
    pltpu.async_copy(table.at[idx_v], grow_v, sem).wait()
    pltpu.sync_copy(grow_v, out_o.at[pl.ds(base, 32)])


def _seg_scratch(d, with_cnt):
    sc = [
        pltpu.VMEM((CH,), jnp.int32),
        pltpu.VMEM((CH,), jnp.int32),
        pltpu.VMEM((CH,), jnp.int32),
        pltpu.VMEM((CH,), jnp.int32),
        pltpu.VMEM((CH, d), F32),
        pltpu.VMEM((CH, d), F32),
        pltpu.VMEM_SHARED((NP, d), F32),
        pltpu.SemaphoreType.DMA,
        pltpu.SemaphoreType.DMA,
    ]
    if with_cnt:
        sc += [pltpu.VMEM((CH,), F32), pltpu.VMEM_SHARED((NP,), F32)]
    return sc


@functools.lru_cache(maxsize=None)
def _segsum(nt, d, e, with_cnt=True):
    assert e % (32 * CH * NBUF) == 0
    n_w = e // (32 * CH)
    dd = d // 16
    if with_cnt:
        out_type = [jax.ShapeDtypeStruct((2 * NP, d), F32),
                    jax.ShapeDtypeStruct((2 * NP,), F32)]
    else:
        out_type = jax.ShapeDtypeStruct((2 * NP, d), F32)

    def body(table, srcl, dstl, out, cnt_out, src0, src1, dst0, dst1,
             rows0, rows1, acc_sh, gs0, gs1, ones_v, cnt_sh):
        c = lax.axis_index("c")
        s = lax.axis_index("s")
        w = s * 2 + c
        _zero_rows(rows0, dd)
        base = s * RPS
        _zero_acc(rows0, base, acc_sh, cnt_sh)
        if ones_v is not None:
            for j in range(CH // 16):
                ones_v[pl.ds(j * 16, 16)] = jnp.ones((16,), F32)
        plsc.subcore_barrier()
        _edge_loop(n_w, w, table, srcl, dstl, (src0, src1), (dst0, dst1),
                   (rows0, rows1), (gs0, gs1), ones_v, acc_sh, cnt_sh)
        plsc.subcore_barrier()
        ob = c * NP + base
        pltpu.sync_copy(acc_sh.at[pl.ds(base, RPS)], out.at[pl.ds(ob, RPS)])
        if cnt_sh is not None:
            pltpu.sync_copy(cnt_sh.at[pl.ds(base, RPS)],
                            cnt_out.at[pl.ds(ob, RPS)])

    kw = dict(mesh=_sc_mesh(), out_type=out_type,
              scratch_types=_seg_scratch(d, with_cnt))
    if with_cnt:
        @functools.partial(pl.kernel, **kw)
        def k(table, srcl, dstl, out, cnt_out, src0, src1, dst0, dst1,
              rows0, rows1, acc_sh, gs0, gs1, ones_v, cnt_sh):
            body(table, srcl, dstl, out, cnt_out, src0, src1, dst0, dst1,
                 rows0, rows1, acc_sh, gs0, gs1, ones_v, cnt_sh)
    else:
        @functools.partial(pl.kernel, **kw)
        def k(table, srcl, dstl, out, src0, src1, dst0, dst1,
              rows0, rows1, acc_sh, gs0, gs1):
            body(table, srcl, dstl, out, None, src0, src1, dst0, dst1,
                 rows0, rows1, acc_sh, gs0, gs1, None, None)

    return k


@functools.lru_cache(maxsize=None)
def _aux1():
    d = 128
    dd = d // 16

    @functools.partial(
        pl.kernel,
        mesh=_sc_mesh(),
        out_type=[
            jax.ShapeDtypeStruct((2 * NP, d), F32),
            jax.ShapeDtypeStruct((2 * NP,), F32),
            jax.ShapeDtypeStruct((1024, d), F32),
            jax.ShapeDtypeStruct((1024, d), F32),
        ],
        scratch_types=_seg_scratch(d, True) + [
            pltpu.VMEM((32,), jnp.int32),
            pltpu.VMEM((32, d), F32),
            pltpu.SemaphoreType.DMA,
        ],
    )
    def k(gen_t, src8, dst8, x_t, idxg, e1_o, k_o, xm_o, gm_o,
          src0, src1, dst0, dst1, rows0, rows1, acc_sh, gs0, gs1,
          ones_v, cnt_sh, idx_v, grow_v, gsem2):
        c = lax.axis_index("c")
        s = lax.axis_index("s")
        w = s * 2 + c
        _gather32(x_t, idxg, xm_o, w, idx_v, grow_v, gsem2)
        _gather32(gen_t, idxg, gm_o, w, idx_v, grow_v, gsem2)
        _zero_rows(rows0, dd)
        base = s * RPS
        _zero_acc(rows0, base, acc_sh, cnt_sh)
        for j in range(CH // 16):
            ones_v[pl.ds(j * 16, 16)] = jnp.ones((16,), F32)
        plsc.subcore_barrier()
        _edge_loop(2, w, gen_t, src8, dst8, (src0, src1), (dst0, dst1),
                   (rows0, rows1), (gs0, gs1), ones_v, acc_sh, cnt_sh)
        plsc.subcore_barrier()
        ob = c * NP + base
        pltpu.sync_copy(acc_sh.at[pl.ds(base, RPS)], e1_o.at[pl.ds(ob, RPS)])
        pltpu.sync_copy(cnt_sh.at[pl.ds(base, RPS)],
                        k_o.at[pl.ds(ob, RPS)])

    return k


@functools.lru_cache(maxsize=None)
def _aux2():
    d = 128
    dd = d // 16

    @functools.partial(
        pl.kernel,
        mesh=_sc_mesh(),
        out_type=[
            jax.ShapeDtypeStruct((2 * NP, d), F32),
            jax.ShapeDtypeStruct((2 * NP, d), F32),
            jax.ShapeDtypeStruct((1024, d), F32),
            jax.ShapeDtypeStruct((1024, d), F32),
        ],
        scratch_types=_seg_scratch(d, False) + [
            pltpu.VMEM((32,), jnp.int32),
            pltpu.VMEM((32, d), F32),
            pltpu.SemaphoreType.DMA,
        ],
    )
    def k(lo_t, hi_t, ar8, dst8, h1lo_t, h1hi_t, idxg,
          e2lo_o, e2hi_o, mlo_o, mhi_o,
          src0, src1, dst0, dst1, rows0, rows1, acc_sh, gs0, gs1,
          idx_v, grow_v, gsem2):
        c = lax.axis_index("c")
        s = lax.axis_index("s")
        w = s * 2 + c
        _gather32(h1lo_t, idxg, mlo_o, w, idx_v, grow_v, gsem2)
        _gather32(h1hi_t, idxg, mhi_o, w, idx_v, grow_v, gsem2)
        base = s * RPS
        ob = c * NP + base
        srcs = (src0, src1)
        dsts = (dst0, dst1)
        rows = (rows0, rows1)
        gsem = (gs0, gs1)
        _zero_rows(rows0, dd)
        _zero_acc(rows0, base, acc_sh, None)
        plsc.subcore_barrier()
        _edge_loop(2, w, lo_t, ar8, dst8, srcs, dsts, rows, gsem, None,
                   acc_sh, None)
        plsc.subcore_barrier()
        pltpu.sync_copy(acc_sh.at[pl.ds(base, RPS)],
                        e2lo_o.at[pl.ds(ob, RPS)])
        _zero_rows(rows0, dd)
        _zero_acc(rows0, base, acc_sh, None)
        plsc.subcore_barrier()
        _edge_loop(2, w, hi_t, ar8, dst8, srcs, dsts, rows, gsem, None,
                   acc_sh, None)
        plsc.subcore_barrier()
        pltpu.sync_copy(acc_sh.at[pl.ds(base, RPS)],
                        e2hi_o.at[pl.ds(ob, RPS)])

    return k


def _trash(num):
    return 10000 + (jnp.arange(num, dtype=jnp.int32) % (NP - 10000))


def _spread_src(num, nt):
    return jnp.arange(num, dtype=jnp.int32) % nt


def _pad_edges(src, dst, e, nt):
    ep = -(-e // 8192) * 8192
    if ep == e:
        return src, dst, e
    pad = ep - e
    src_p = jnp.concatenate([src, _spread_src(pad, nt)])
    dst_p = jnp.concatenate([dst, _trash(pad)])
    return src_p, dst_p, ep


def _mm(a, w):
    return jnp.dot(a, w, preferred_element_type=F32)


def _gen_body(aggA, aggB, cntA, cntB, xb, wgl, bgl, wgr, wd1, bd1, wd2, bd2,
              gen_o):
    cnt = cntA[...] + cntB[...]
    mean0 = (aggA[...] + aggB[...]) / jnp.maximum(cnt, 1.0)
    h = jnp.maximum(_mm(mean0, wgl[...]) + bgl[...] + _mm(xb[...], wgr[...]),
                    0.0)
    t = jnp.maximum(_mm(h, wd1[...]) + bd1[...], 0.0)
    gen_o[...] = _mm(t, wd2[...]) + bd2[...]


def _conv1_body(aggA, aggB, e1A, e1B, cntA, cntB, kA, kB, xb, wl1, bl1, wr1,
                h1lo_o, h1hi_o, den_o):
    den = jnp.maximum(cntA[...] + cntB[...] + kA[...] + kB[...], 1.0)
    den_r = 1.0 / den
    mean1 = (aggA[...] + aggB[...] + e1A[...] + e1B[...]) * den_r
    h1 = jnp.maximum(_mm(mean1, wl1[...]) + bl1[...] + _mm(xb[...], wr1[...]),
                     0.0)
    h1lo_o[...] = h1[:, :128]
    h1hi_o[...] = h1[:, 128:]
    den_o[...] = den_r


def _new1_body(xm, gm, wl1, bl1, wr1, lo_o, hi_o):
    h1n = jnp.maximum(_mm(xm[...], wl1[...]) + bl1[...] +
                      _mm(gm[...], wr1[...]), 0.0)
    lo_o[...] = h1n[:, :128]
    hi_o[...] = h1n[:, 128:]


def _conv2_body(aloA, aloB, ahiA, ahiB, eloA, eloB, ehiA, ehiB, den, h1lo,
                h1hi, wl2, bl2, wr2, wp, bp, out_o):
    d = den[...]
    mlo = (aloA[...] + aloB[...] + eloA[...] + eloB[...]) * d
    mhi = (ahiA[...] + ahiB[...] + ehiA[...] + ehiB[...]) * d
    wl2v = wl2[...]
    wr2v = wr2[...]
    h2 = jnp.maximum(
        _mm(mlo, wl2v[:128]) + _mm(mhi, wl2v[128:]) + bl2[...] +
        _mm(h1lo[...], wr2v[:128]) + _mm(h1hi[...], wr2v[128:]), 0.0)
    out_o[...] = _mm(h2, wp[...]) + bp[...]


def _new2_body(h1mlo, h1mhi, h1nlo, h1nhi, wl2, bl2, wr2, wp, bp, out_o):
    wl2v = wl2[...]
    wr2v = wr2[...]
    h2n = jnp.maximum(
        _mm(h1mlo[...], wl2v[:128]) + _mm(h1mhi[...], wl2v[128:]) + bl2[...] +
        _mm(h1nlo[...], wr2v[:128]) + _mm(h1nhi[...], wr2v[128:]), 0.0)
    out_o[...] = _mm(h2n, wp[...]) + bp[...]


def _row_spec(w, two_part):
    nb = NP // BLK
    if two_part == 0:
        return pl.BlockSpec((BLK, w), lambda i: (i, 0))
    return pl.BlockSpec((BLK, w), lambda i, nb=nb: (i + nb, 0))


def _full_spec(shape):
    nd = len(shape)
    return pl.BlockSpec(shape, lambda i: (0,) * nd)


def kernel(x, edge_index, missing_indices, Wl1, bl1, Wr1, Wl2, bl2, Wr2,
           Wp, bp, Wgl, bgl, Wgr, Wd1, bd1, Wd2, bd2):
    n, dx = x.shape
    e = edge_index.shape[1]
    m = missing_indices.shape[0]
    src = edge_index[0].astype(jnp.int32)
    dst = edge_index[1].astype(jnp.int32)
    midx = missing_indices.astype(jnp.int32)
    mp = 1024
    x_pad = jnp.pad(x, ((0, NP - n), (0, 0)))
    src_p, dst_p, ep = _pad_edges(src, dst, e, n)
    src_m = jnp.concatenate([midx, jnp.zeros((mp - m,), jnp.int32)])
    src_m8 = jnp.concatenate([midx, _spread_src(8192 - m, n)])
    dst_m8 = jnp.concatenate([midx, _trash(8192 - m)])
    ar8 = jnp.concatenate([jnp.arange(mp, dtype=jnp.int32),
                           _spread_src(8192 - mp, mp)])

    bgl_r = bgl.reshape(1, -1)
    bd1_r = bd1.reshape(1, -1)
    bd2_r = bd2.reshape(1, -1)
    bl1_r = bl1.reshape(1, -1)
    bl2_r = bl2.reshape(1, -1)
    bp_r = bp.reshape(1, -1)

    agg, cnt = _segsum(NP, 128, ep)(x_pad, src_p, dst_p)
    cnt2 = cnt.reshape(2 * NP, 1)

    nb = NP // BLK
    gen = pl.pallas_call(
        _gen_body,
        grid=(nb,),
        in_specs=[
            _row_spec(128, 0), _row_spec(128, 1),
            _row_spec(1, 0), _row_spec(1, 1),
            _row_spec(128, 0),
            _full_spec((128, 256)), _full_spec((1, 256)),
            _full_spec((128, 256)),
            _full_spec((256, 256)), _full_spec((1, 256)),
            _full_spec((256, 128)), _full_spec((1, 128)),
        ],
        out_specs=_row_spec(128, 0),
        out_shape=jax.ShapeDtypeStruct((NP, 128), F32),
    )(agg, agg, cnt2, cnt2, x_pad, Wgl, bgl_r, Wgr, Wd1, bd1_r, Wd2, bd2_r)

    e1, kcnt, xm, gm = _aux1()(gen, src_m8, dst_m8, x_pad, src_m)
    k2 = kcnt.reshape(2 * NP, 1)

    h1lo, h1hi, den_r = pl.pallas_call(
        _conv1_body,
        grid=(nb,),
        in_specs=[
            _row_spec(128, 0), _row_spec(128, 1),
            _row_spec(128, 0), _row_spec(128, 1),
            _row_spec(1, 0), _row_spec(1, 1),
            _row_spec(1, 0), _row_spec(1, 1),
            _row_spec(128, 0),
            _full_spec((128, 256)), _full_spec((1, 256)),
            _full_spec((128, 256)),
        ],
        out_specs=[_row_spec(128, 0), _row_spec(128, 0), _row_spec(1, 0)],
        out_shape=[
            jax.ShapeDtypeStruct((NP, 128), F32),
            jax.ShapeDtypeStruct((NP, 128), F32),
            jax.ShapeDtypeStruct((NP, 1), F32),
        ],
    )(agg, agg, e1, e1, cnt2, cnt2, k2, k2, x_pad, Wl1, bl1_r, Wr1)

    h1nlo, h1nhi = pl.pallas_call(
        _new1_body,
        grid=(mp // BLK,),
        in_specs=[
            _row_spec(128, 0), _row_spec(128, 0),
            _full_spec((128, 256)), _full_spec((1, 256)),
            _full_spec((128, 256)),
        ],
        out_specs=[_row_spec(128, 0), _row_spec(128, 0)],
        out_shape=[
            jax.ShapeDtypeStruct((mp, 128), F32),
            jax.ShapeDtypeStruct((mp, 128), F32),
        ],
    )(xm, gm, Wl1, bl1_r, Wr1)

    a2lo = _segsum(NP, 128, ep, False)(h1lo, src_p, dst_p)
    a2hi = _segsum(NP, 128, ep, False)(h1hi, src_p, dst_p)
    e2lo, e2hi, h1mlo, h1mhi = _aux2()(h1nlo, h1nhi, ar8, dst_m8,
                                       h1lo, h1hi, src_m)

    out_main = pl.pallas_call(
        _conv2_body,
        grid=(nb,),
        in_specs=[
            _row_spec(128, 0), _row_spec(128, 1),
            _row_spec(128, 0), _row_spec(128, 1),
            _row_spec(128, 0), _row_spec(128, 1),
            _row_spec(128, 0), _row_spec(128, 1),
            _row_spec(1, 0),
            _row_spec(128, 0), _row_spec(128, 0),
            _full_spec((256, 256)), _full_spec((1, 256)),
            _full_spec((256, 256)),
            _full_spec((256, 64)), _full_spec((1, 64)),
        ],
        out_specs=_row_spec(64, 0),
        out_shape=jax.ShapeDtypeStruct((NP, 64), F32),
    )(a2lo, a2lo, a2hi, a2hi, e2lo, e2lo, e2hi, e2hi, den_r, h1lo, h1hi,
      Wl2, bl2_r, Wr2, Wp, bp_r)

    out_new = pl.pallas_call(
        _new2_body,
        grid=(mp // BLK,),
        in_specs=[
            _row_spec(128, 0), _row_spec(128, 0),
            _row_spec(128, 0), _row_spec(128, 0),
            _full_spec((256, 256)), _full_spec((1, 256)),
            _full_spec((256, 256)),
            _full_spec((256, 64)), _full_spec((1, 64)),
        ],
        out_specs=_row_spec(64, 0),
        out_shape=jax.ShapeDtypeStruct((mp, 64), F32),
    )(h1mlo, h1mhi, h1nlo, h1nhi, Wl2, bl2_r, Wr2, Wp, bp_r)

    return jnp.concatenate([out_main[:n], out_new[:m]], axis=0)


# speedup vs baseline: 3.6261x; 1.0844x over previous
import functools

import jax
import jax.numpy as jnp
from jax import lax
from jax.experimental import pallas as pl
from jax.experimental.pallas import tpu as pltpu
from jax.experimental.pallas import tpu_sc as plsc

NP = 10240
RPS = NP // 16
TRASH = NP - 1
CH = 128
BLK = 256
F32 = jnp.float32


def _sc_mesh():
    return plsc.VectorSubcoreMesh(core_axis_name="c", subcore_axis_name="s")


NBUF = 2


def _zero_rows(rows0, dd):
    def zero_body(i, carry):
        rows0[i // dd, pl.ds((i % dd) * 16, 16)] = jnp.zeros((16,), F32)
        return carry

    lax.fori_loop(0, CH * dd, zero_body, 0)


def _zero_acc(rows0, base, acc_sh, cnt_sh):
    for j in range(RPS // CH):
        pltpu.sync_copy(rows0, acc_sh.at[pl.ds(base + j * CH, CH)])
        if cnt_sh is not None:
            pltpu.sync_copy(rows0.at[0], cnt_sh.at[pl.ds(base + j * CH, CH)])


def _edge_loop(n_w, w, table, srcl, dstl, srcs, dsts, rows, gsem, ones_v,
               acc_sh, cnt_sh):
    def load_and_fire(b, i):
        bb = (w + i * 32) * CH
        pltpu.sync_copy(srcl.at[pl.ds(bb, CH)], srcs[b])
        pltpu.sync_copy(dstl.at[pl.ds(bb, CH)], dsts[b])
        pltpu.async_copy(table.at[srcs[b]], rows[b], gsem[b])

    def drain_and_scatter(b):
        pltpu.make_async_copy(table.at[srcs[b]], rows[b], gsem[b]).wait()
        pltpu.sync_copy(rows[b], acc_sh.at[dsts[b]], add=True)
        if cnt_sh is not None:
            pltpu.sync_copy(ones_v, cnt_sh.at[dsts[b]], add=True)

    for b in range(NBUF):
        load_and_fire(b, b)

    def body(g, carry):
        for b in range(NBUF):
            i = g * NBUF + b
            drain_and_scatter(b)
            load_and_fire(b, i + NBUF)
        return carry

    lax.fori_loop(0, (n_w - NBUF) // NBUF, body, 0)
    for b in range(NBUF):
        drain_and_scatter(b)


def _gather32(table, idxg, out_o, w, idx_v, grow_v, sem):
    base = w * 32
    pltpu.sync_copy(idxg.at[pl.ds(base, 32)], idx_v)
    pltpu.async_copy(table.at[idx_v], grow_v, sem).wait()
    pltpu.sync_copy(grow_v, out_o.at[pl.ds(base, 32)])


def _seg_scratch(d, with_cnt):
    sc = [
        pltpu.VMEM((CH,), jnp.int32),
        pltpu.VMEM((CH,), jnp.int32),
        pltpu.VMEM((CH,), jnp.int32),
        pltpu.VMEM((CH,), jnp.int32),
        pltpu.VMEM((CH, d), F32),
        pltpu.VMEM((CH, d), F32),
        pltpu.VMEM_SHARED((NP, d), F32),
        pltpu.SemaphoreType.DMA,
        pltpu.SemaphoreType.DMA,
    ]
    if with_cnt:
        sc += [pltpu.VMEM((CH,), F32), pltpu.VMEM_SHARED((NP,), F32)]
    return sc


SUP = 8


def _sup_scratch(d, with_cnt):
    sc = [
        pltpu.VMEM((SUP, CH), jnp.int32),
        pltpu.VMEM((SUP, CH), jnp.int32),
        pltpu.VMEM((CH, d), F32),
        pltpu.VMEM((CH, d), F32),
        pltpu.VMEM_SHARED((NP, d), F32),
        pltpu.SemaphoreType.DMA,
        pltpu.SemaphoreType.DMA,
    ]
    if with_cnt:
        sc += [pltpu.VMEM((CH,), F32), pltpu.VMEM_SHARED((NP,), F32)]
    return sc


@functools.lru_cache(maxsize=None)
def _segsum_sup(nt, d, e, with_cnt=True):
    assert e % (32 * CH * SUP) == 0
    n_sup = e // (32 * CH * SUP)
    dd = d // 16
    if with_cnt:
        out_type = [jax.ShapeDtypeStruct((2 * NP, d), F32),
                    jax.ShapeDtypeStruct((2 * NP,), F32)]
    else:
        out_type = jax.ShapeDtypeStruct((2 * NP, d), F32)

    def body_fn(table, src2d, dst2d, out, cnt_out, src_sup, dst_sup,
                rows0, rows1, acc_sh, gs0, gs1, ones_v, cnt_sh):
        rows = (rows0, rows1)
        gsem = (gs0, gs1)
        c = lax.axis_index("c")
        s = lax.axis_index("s")
        w = s * 2 + c
        _zero_rows(rows0, dd)
        base = s * RPS
        _zero_acc(rows0, base, acc_sh, cnt_sh)
        if ones_v is not None:
            for j in range(CH // 16):
                ones_v[pl.ds(j * 16, 16)] = jnp.ones((16,), F32)
        plsc.subcore_barrier()

        def gfire(j):
            pltpu.async_copy(table.at[src_sup.at[j]], rows[j % 2],
                             gsem[j % 2])

        def gdrain_scatter(j):
            pltpu.make_async_copy(table.at[src_sup.at[j]], rows[j % 2],
                                  gsem[j % 2]).wait()
            pltpu.sync_copy(rows[j % 2], acc_sh.at[dst_sup.at[j]], add=True)
            if cnt_sh is not None:
                pltpu.sync_copy(ones_v, cnt_sh.at[dst_sup.at[j]], add=True)

        def body(q, carry):
            row0 = (w + q * 32) * SUP
            pltpu.sync_copy(src2d.at[pl.ds(row0, SUP)], src_sup)
            pltpu.sync_copy(dst2d.at[pl.ds(row0, SUP)], dst_sup)
            gfire(0)
            for j in range(1, SUP):
                gfire(j)
                gdrain_scatter(j - 1)
            gdrain_scatter(SUP - 1)
            return carry

        lax.fori_loop(0, n_sup, body, 0)
        plsc.subcore_barrier()
        ob = c * NP + base
        pltpu.sync_copy(acc_sh.at[pl.ds(base, RPS)], out.at[pl.ds(ob, RPS)])
        if cnt_sh is not None:
            pltpu.sync_copy(cnt_sh.at[pl.ds(base, RPS)],
                            cnt_out.at[pl.ds(ob, RPS)])

    kw = dict(mesh=_sc_mesh(), out_type=out_type,
              scratch_types=_sup_scratch(d, with_cnt))
    if with_cnt:
        @functools.partial(pl.kernel, **kw)
        def k(table, src2d, dst2d, out, cnt_out, src_sup, dst_sup,
              rows0, rows1, acc_sh, gs0, gs1, ones_v, cnt_sh):
            body_fn(table, src2d, dst2d, out, cnt_out, src_sup, dst_sup,
                    rows0, rows1, acc_sh, gs0, gs1, ones_v, cnt_sh)
    else:
        @functools.partial(pl.kernel, **kw)
        def k(table, src2d, dst2d, out, src_sup, dst_sup,
              rows0, rows1, acc_sh, gs0, gs1):
            body_fn(table, src2d, dst2d, out, None, src_sup, dst_sup,
                    rows0, rows1, acc_sh, gs0, gs1, None, None)

    return k


@functools.lru_cache(maxsize=None)
def _aux1():
    d = 128
    dd = d // 16

    @functools.partial(
        pl.kernel,
        mesh=_sc_mesh(),
        out_type=[
            jax.ShapeDtypeStruct((2 * NP, d), F32),
            jax.ShapeDtypeStruct((2 * NP,), F32),
            jax.ShapeDtypeStruct((1024, d), F32),
            jax.ShapeDtypeStruct((1024, d), F32),
        ],
        scratch_types=_seg_scratch(d, True) + [
            pltpu.VMEM((32,), jnp.int32),
            pltpu.VMEM((32, d), F32),
            pltpu.SemaphoreType.DMA,
        ],
    )
    def k(gen_t, src8, dst8, x_t, idxg, e1_o, k_o, xm_o, gm_o,
          src0, src1, dst0, dst1, rows0, rows1, acc_sh, gs0, gs1,
          ones_v, cnt_sh, idx_v, grow_v, gsem2):
        c = lax.axis_index("c")
        s = lax.axis_index("s")
        w = s * 2 + c
        _gather32(x_t, idxg, xm_o, w, idx_v, grow_v, gsem2)
        _gather32(gen_t, idxg, gm_o, w, idx_v, grow_v, gsem2)
        _zero_rows(rows0, dd)
        base = s * RPS
        _zero_acc(rows0, base, acc_sh, cnt_sh)
        for j in range(CH // 16):
            ones_v[pl.ds(j * 16, 16)] = jnp.ones((16,), F32)
        plsc.subcore_barrier()
        _edge_loop(2, w, gen_t, src8, dst8, (src0, src1), (dst0, dst1),
                   (rows0, rows1), (gs0, gs1), ones_v, acc_sh, cnt_sh)
        plsc.subcore_barrier()
        ob = c * NP + base
        pltpu.sync_copy(acc_sh.at[pl.ds(base, RPS)], e1_o.at[pl.ds(ob, RPS)])
        pltpu.sync_copy(cnt_sh.at[pl.ds(base, RPS)],
                        k_o.at[pl.ds(ob, RPS)])

    return k


@functools.lru_cache(maxsize=None)
def _aux2():
    d = 128
    dd = d // 16

    @functools.partial(
        pl.kernel,
        mesh=_sc_mesh(),
        out_type=[
            jax.ShapeDtypeStruct((2 * NP, d), F32),
            jax.ShapeDtypeStruct((2 * NP, d), F32),
            jax.ShapeDtypeStruct((1024, d), F32),
            jax.ShapeDtypeStruct((1024, d), F32),
        ],
        scratch_types=_seg_scratch(d, False) + [
            pltpu.VMEM((32,), jnp.int32),
            pltpu.VMEM((32, d), F32),
            pltpu.SemaphoreType.DMA,
        ],
    )
    def k(lo_t, hi_t, ar8, dst8, h1lo_t, h1hi_t, idxg,
          e2lo_o, e2hi_o, mlo_o, mhi_o,
          src0, src1, dst0, dst1, rows0, rows1, acc_sh, gs0, gs1,
          idx_v, grow_v, gsem2):
        c = lax.axis_index("c")
        s = lax.axis_index("s")
        w = s * 2 + c
        _gather32(h1lo_t, idxg, mlo_o, w, idx_v, grow_v, gsem2)
        _gather32(h1hi_t, idxg, mhi_o, w, idx_v, grow_v, gsem2)
        base = s * RPS
        ob = c * NP + base
        srcs = (src0, src1)
        dsts = (dst0, dst1)
        rows = (rows0, rows1)
        gsem = (gs0, gs1)
        _zero_rows(rows0, dd)
        _zero_acc(rows0, base, acc_sh, None)
        plsc.subcore_barrier()
        _edge_loop(2, w, lo_t, ar8, dst8, srcs, dsts, rows, gsem, None,
                   acc_sh, None)
        plsc.subcore_barrier()
        pltpu.sync_copy(acc_sh.at[pl.ds(base, RPS)],
                        e2lo_o.at[pl.ds(ob, RPS)])
        _zero_rows(rows0, dd)
        _zero_acc(rows0, base, acc_sh, None)
        plsc.subcore_barrier()
        _edge_loop(2, w, hi_t, ar8, dst8, srcs, dsts, rows, gsem, None,
                   acc_sh, None)
        plsc.subcore_barrier()
        pltpu.sync_copy(acc_sh.at[pl.ds(base, RPS)],
                        e2hi_o.at[pl.ds(ob, RPS)])

    return k


def _trash(num):
    return 10000 + (jnp.arange(num, dtype=jnp.int32) % (NP - 10000))


def _spread_src(num, nt):
    return jnp.arange(num, dtype=jnp.int32) % nt


def _pad_edges(src, dst, e, nt):
    ep = -(-e // (32 * CH * SUP)) * (32 * CH * SUP)
    if ep == e:
        return src, dst, e
    pad = ep - e
    src_p = jnp.concatenate([src, _spread_src(pad, nt)])
    dst_p = jnp.concatenate([dst, _trash(pad)])
    return src_p, dst_p, ep


def _mm(a, w):
    return jnp.dot(a, w, preferred_element_type=F32)


def _gen_body(aggA, aggB, cntA, cntB, xb, wgl, bgl, wgr, wd1, bd1, wd2, bd2,
              gen_o):
    cnt = cntA[...] + cntB[...]
    mean0 = (aggA[...] + aggB[...]) / jnp.maximum(cnt, 1.0)
    h = jnp.maximum(_mm(mean0, wgl[...]) + bgl[...] + _mm(xb[...], wgr[...]),
                    0.0)
    t = jnp.maximum(_mm(h, wd1[...]) + bd1[...], 0.0)
    gen_o[...] = _mm(t, wd2[...]) + bd2[...]


def _conv1_body(aggA, aggB, e1A, e1B, cntA, cntB, kA, kB, xb, wl1, bl1, wr1,
                h1lo_o, h1hi_o, den_o):
    den = jnp.maximum(cntA[...] + cntB[...] + kA[...] + kB[...], 1.0)
    den_r = 1.0 / den
    mean1 = (aggA[...] + aggB[...] + e1A[...] + e1B[...]) * den_r
    h1 = jnp.maximum(_mm(mean1, wl1[...]) + bl1[...] + _mm(xb[...], wr1[...]),
                     0.0)
    h1lo_o[...] = h1[:, :128]
    h1hi_o[...] = h1[:, 128:]
    den_o[...] = den_r


def _new1_body(xm, gm, wl1, bl1, wr1, lo_o, hi_o):
    h1n = jnp.maximum(_mm(xm[...], wl1[...]) + bl1[...] +
                      _mm(gm[...], wr1[...]), 0.0)
    lo_o[...] = h1n[:, :128]
    hi_o[...] = h1n[:, 128:]


def _conv2_body(aloA, aloB, ahiA, ahiB, eloA, eloB, ehiA, ehiB, den, h1lo,
                h1hi, wl2, bl2, wr2, wp, bp, out_o):
    d = den[...]
    mlo = (aloA[...] + aloB[...] + eloA[...] + eloB[...]) * d
    mhi = (ahiA[...] + ahiB[...] + ehiA[...] + ehiB[...]) * d
    wl2v = wl2[...]
    wr2v = wr2[...]
    h2 = jnp.maximum(
        _mm(mlo, wl2v[:128]) + _mm(mhi, wl2v[128:]) + bl2[...] +
        _mm(h1lo[...], wr2v[:128]) + _mm(h1hi[...], wr2v[128:]), 0.0)
    out_o[...] = _mm(h2, wp[...]) + bp[...]


def _new2_body(h1mlo, h1mhi, h1nlo, h1nhi, wl2, bl2, wr2, wp, bp, out_o):
    wl2v = wl2[...]
    wr2v = wr2[...]
    h2n = jnp.maximum(
        _mm(h1mlo[...], wl2v[:128]) + _mm(h1mhi[...], wl2v[128:]) + bl2[...] +
        _mm(h1nlo[...], wr2v[:128]) + _mm(h1nhi[...], wr2v[128:]), 0.0)
    out_o[...] = _mm(h2n, wp[...]) + bp[...]


def _row_spec(w, two_part):
    nb = NP // BLK
    if two_part == 0:
        return pl.BlockSpec((BLK, w), lambda i: (i, 0))
    return pl.BlockSpec((BLK, w), lambda i, nb=nb: (i + nb, 0))


def _full_spec(shape):
    nd = len(shape)
    return pl.BlockSpec(shape, lambda i: (0,) * nd)


def kernel(x, edge_index, missing_indices, Wl1, bl1, Wr1, Wl2, bl2, Wr2,
           Wp, bp, Wgl, bgl, Wgr, Wd1, bd1, Wd2, bd2):
    n, dx = x.shape
    e = edge_index.shape[1]
    m = missing_indices.shape[0]
    src = edge_index[0].astype(jnp.int32)
    dst = edge_index[1].astype(jnp.int32)
    midx = missing_indices.astype(jnp.int32)
    mp = 1024
    x_pad = jnp.pad(x, ((0, NP - n), (0, 0)))
    src_p, dst_p, ep = _pad_edges(src, dst, e, n)
    src_m = jnp.concatenate([midx, jnp.zeros((mp - m,), jnp.int32)])
    src_m8 = jnp.concatenate([midx, _spread_src(8192 - m, n)])
    dst_m8 = jnp.concatenate([midx, _trash(8192 - m)])
    ar8 = jnp.concatenate([jnp.arange(mp, dtype=jnp.int32),
                           _spread_src(8192 - mp, mp)])

    bgl_r = bgl.reshape(1, -1)
    bd1_r = bd1.reshape(1, -1)
    bd2_r = bd2.reshape(1, -1)
    bl1_r = bl1.reshape(1, -1)
    bl2_r = bl2.reshape(1, -1)
    bp_r = bp.reshape(1, -1)

    src2 = src_p.reshape(-1, CH)
    dst2 = dst_p.reshape(-1, CH)
    agg, cnt = _segsum_sup(NP, 128, ep)(x_pad, src2, dst2)
    cnt2 = cnt.reshape(2 * NP, 1)

    nb = NP // BLK
    gen = pl.pallas_call(
        _gen_body,
        grid=(nb,),
        in_specs=[
            _row_spec(128, 0), _row_spec(128, 1),
            _row_spec(1, 0), _row_spec(1, 1),
            _row_spec(128, 0),
            _full_spec((128, 256)), _full_spec((1, 256)),
            _full_spec((128, 256)),
            _full_spec((256, 256)), _full_spec((1, 256)),
            _full_spec((256, 128)), _full_spec((1, 128)),
        ],
        out_specs=_row_spec(128, 0),
        out_shape=jax.ShapeDtypeStruct((NP, 128), F32),
    )(agg, agg, cnt2, cnt2, x_pad, Wgl, bgl_r, Wgr, Wd1, bd1_r, Wd2, bd2_r)

    e1, kcnt, xm, gm = _aux1()(gen, src_m8, dst_m8, x_pad, src_m)
    k2 = kcnt.reshape(2 * NP, 1)

    h1lo, h1hi, den_r = pl.pallas_call(
        _conv1_body,
        grid=(nb,),
        in_specs=[
            _row_spec(128, 0), _row_spec(128, 1),
            _row_spec(128, 0), _row_spec(128, 1),
            _row_spec(1, 0), _row_spec(1, 1),
            _row_spec(1, 0), _row_spec(1, 1),
            _row_spec(128, 0),
            _full_spec((128, 256)), _full_spec((1, 256)),
            _full_spec((128, 256)),
        ],
        out_specs=[_row_spec(128, 0), _row_spec(128, 0), _row_spec(1, 0)],
        out_shape=[
            jax.ShapeDtypeStruct((NP, 128), F32),
            jax.ShapeDtypeStruct((NP, 128), F32),
            jax.ShapeDtypeStruct((NP, 1), F32),
        ],
    )(agg, agg, e1, e1, cnt2, cnt2, k2, k2, x_pad, Wl1, bl1_r, Wr1)

    h1nlo, h1nhi = pl.pallas_call(
        _new1_body,
        grid=(mp // BLK,),
        in_specs=[
            _row_spec(128, 0), _row_spec(128, 0),
            _full_spec((128, 256)), _full_spec((1, 256)),
            _full_spec((128, 256)),
        ],
        out_specs=[_row_spec(128, 0), _row_spec(128, 0)],
        out_shape=[
            jax.ShapeDtypeStruct((mp, 128), F32),
            jax.ShapeDtypeStruct((mp, 128), F32),
        ],
    )(xm, gm, Wl1, bl1_r, Wr1)

    a2lo = _segsum_sup(NP, 128, ep, False)(h1lo, src2, dst2)
    a2hi = _segsum_sup(NP, 128, ep, False)(h1hi, src2, dst2)
    e2lo, e2hi, h1mlo, h1mhi = _aux2()(h1nlo, h1nhi, ar8, dst_m8,
                                       h1lo, h1hi, src_m)

    out_main = pl.pallas_call(
        _conv2_body,
        grid=(nb,),
        in_specs=[
            _row_spec(128, 0), _row_spec(128, 1),
            _row_spec(128, 0), _row_spec(128, 1),
            _row_spec(128, 0), _row_spec(128, 1),
            _row_spec(128, 0), _row_spec(128, 1),
            _row_spec(1, 0),
            _row_spec(128, 0), _row_spec(128, 0),
            _full_spec((256, 256)), _full_spec((1, 256)),
            _full_spec((256, 256)),
            _full_spec((256, 64)), _full_spec((1, 64)),
        ],
        out_specs=_row_spec(64, 0),
        out_shape=jax.ShapeDtypeStruct((NP, 64), F32),
    )(a2lo, a2lo, a2hi, a2hi, e2lo, e2lo, e2hi, e2hi, den_r, h1lo, h1hi,
      Wl2, bl2_r, Wr2, Wp, bp_r)

    out_new = pl.pallas_call(
        _new2_body,
        grid=(mp // BLK,),
        in_specs=[
            _row_spec(128, 0), _row_spec(128, 0),
            _row_spec(128, 0), _row_spec(128, 0),
            _full_spec((256, 256)), _full_spec((1, 256)),
            _full_spec((256, 256)),
            _full_spec((256, 64)), _full_spec((1, 64)),
        ],
        out_specs=_row_spec(64, 0),
        out_shape=jax.ShapeDtypeStruct((mp, 64), F32),
    )(h1mlo, h1mhi, h1nlo, h1nhi, Wl2, bl2_r, Wr2, Wp, bp_r)

    return jnp.concatenate([out_main[:n], out_new[:m]], axis=0)


# speedup vs baseline: 3.6483x; 1.0061x over previous
import functools

import jax
import jax.numpy as jnp
from jax import lax
from jax.experimental import pallas as pl
from jax.experimental.pallas import tpu as pltpu
from jax.experimental.pallas import tpu_sc as plsc

NP = 10240
RPS = NP // 16
TRASH = NP - 1
CH = 128
BLK = 256
F32 = jnp.float32


def _sc_mesh():
    return plsc.VectorSubcoreMesh(core_axis_name="c", subcore_axis_name="s")


NBUF = 2


def _zero_rows(rows0, dd):
    def zero_body(i, carry):
        rows0[i // dd, pl.ds((i % dd) * 16, 16)] = jnp.zeros((16,), F32)
        return carry

    lax.fori_loop(0, CH * dd, zero_body, 0)


def _zero_acc(rows0, base, acc_sh, cnt_sh):
    for j in range(RPS // CH):
        pltpu.sync_copy(rows0, acc_sh.at[pl.ds(base + j * CH, CH)])
        if cnt_sh is not None:
            pltpu.sync_copy(rows0.at[0], cnt_sh.at[pl.ds(base + j * CH, CH)])


def _edge_loop(n_w, w, table, srcl, dstl, srcs, dsts, rows, gsem, ones_v,
               acc_sh, cnt_sh):
    def load_and_fire(b, i):
        bb = (w + i * 32) * CH
        pltpu.sync_copy(srcl.at[pl.ds(bb, CH)], srcs[b])
        pltpu.sync_copy(dstl.at[pl.ds(bb, CH)], dsts[b])
        pltpu.async_copy(table.at[srcs[b]], rows[b], gsem[b])

    def drain_and_scatter(b):
        pltpu.make_async_copy(table.at[srcs[b]], rows[b], gsem[b]).wait()
        pltpu.sync_copy(rows[b], acc_sh.at[dsts[b]], add=True)
        if cnt_sh is not None:
            pltpu.sync_copy(ones_v, cnt_sh.at[dsts[b]], add=True)

    for b in range(NBUF):
        load_and_fire(b, b)

    def body(g, carry):
        for b in range(NBUF):
            i = g * NBUF + b
            drain_and_scatter(b)
            load_and_fire(b, i + NBUF)
        return carry

    lax.fori_loop(0, (n_w - NBUF) // NBUF, body, 0)
    for b in range(NBUF):
        drain_and_scatter(b)


def _gather32(table, idxg, out_o, w, idx_v, grow_v, sem):
    base = w * 32
    pltpu.sync_copy(idxg.at[pl.ds(base, 32)], idx_v)
    pltpu.async_copy(table.at[idx_v], grow_v, sem).wait()
    pltpu.sync_copy(grow_v, out_o.at[pl.ds(base, 32)])


def _seg_scratch(d, with_cnt):
    sc = [
        pltpu.VMEM((CH,), jnp.int32),
        pltpu.VMEM((CH,), jnp.int32),
        pltpu.VMEM((CH,), jnp.int32),
        pltpu.VMEM((CH,), jnp.int32),
        pltpu.VMEM((CH, d), F32),
        pltpu.VMEM((CH, d), F32),
        pltpu.VMEM_SHARED((NP, d), F32),
        pltpu.SemaphoreType.DMA,
        pltpu.SemaphoreType.DMA,
    ]
    if with_cnt:
        sc += [pltpu.VMEM((CH,), F32), pltpu.VMEM_SHARED((NP,), F32)]
    return sc


SUP = 8


def _sup_scratch(d, with_cnt):
    sc = [
        pltpu.VMEM((SUP, CH), jnp.int32),
        pltpu.VMEM((SUP, CH), jnp.int32),
        pltpu.VMEM((CH, d), F32),
        pltpu.VMEM((CH, d), F32),
        pltpu.VMEM_SHARED((NP, d), F32),
        pltpu.SemaphoreType.DMA,
        pltpu.SemaphoreType.DMA,
        pltpu.SemaphoreType.DMA,
        pltpu.SemaphoreType.DMA,
    ]
    if with_cnt:
        sc += [pltpu.VMEM((CH,), F32), pltpu.VMEM_SHARED((NP,), F32)]
    return sc


@functools.lru_cache(maxsize=None)
def _segsum_sup(nt, d, e, with_cnt=True):
    assert e % (32 * CH * SUP) == 0
    n_sup = e // (32 * CH * SUP)
    dd = d // 16
    if with_cnt:
        out_type = [jax.ShapeDtypeStruct((2 * NP, d), F32),
                    jax.ShapeDtypeStruct((2 * NP,), F32)]
    else:
        out_type = jax.ShapeDtypeStruct((2 * NP, d), F32)

    def body_fn(table, src2d, dst2d, out, cnt_out, src_sup, dst_sup,
                rows0, rows1, acc_sh, gs0, gs1, ss0, ss1, ones_v, cnt_sh):
        rows = (rows0, rows1)
        gsem = (gs0, gs1)
        ssem = (ss0, ss1)
        c = lax.axis_index("c")
        s = lax.axis_index("s")
        w = s * 2 + c
        _zero_rows(rows0, dd)
        base = s * RPS
        _zero_acc(rows0, base, acc_sh, cnt_sh)
        if ones_v is not None:
            for j in range(CH // 16):
                ones_v[pl.ds(j * 16, 16)] = jnp.ones((16,), F32)
        plsc.subcore_barrier()

        def gfire(j):
            pltpu.async_copy(table.at[src_sup.at[j]], rows[j % 2],
                             gsem[j % 2])

        def gdrain(j):
            pltpu.make_async_copy(table.at[src_sup.at[j]], rows[j % 2],
                                  gsem[j % 2]).wait()

        def sfire(j):
            pltpu.async_copy(rows[j % 2], acc_sh.at[dst_sup.at[j]],
                             ssem[j % 2], add=True)
            if cnt_sh is not None:
                pltpu.sync_copy(ones_v, cnt_sh.at[dst_sup.at[j]], add=True)

        def swait(j):
            pltpu.make_async_copy(rows[j % 2], acc_sh.at[dst_sup.at[j]],
                                  ssem[j % 2]).wait()

        def body(q, carry):
            row0 = (w + q * 32) * SUP
            pltpu.sync_copy(src2d.at[pl.ds(row0, SUP)], src_sup)
            pltpu.sync_copy(dst2d.at[pl.ds(row0, SUP)], dst_sup)
            gfire(0)
            for j in range(SUP):
                if j + 1 < SUP:
                    if j >= 1:
                        swait(j - 1)
                    gfire(j + 1)
                gdrain(j)
                sfire(j)
            swait(SUP - 2)
            swait(SUP - 1)
            return carry

        lax.fori_loop(0, n_sup, body, 0)
        plsc.subcore_barrier()
        ob = c * NP + base
        pltpu.sync_copy(acc_sh.at[pl.ds(base, RPS)], out.at[pl.ds(ob, RPS)])
        if cnt_sh is not None:
            pltpu.sync_copy(cnt_sh.at[pl.ds(base, RPS)],
                            cnt_out.at[pl.ds(ob, RPS)])

    kw = dict(mesh=_sc_mesh(), out_type=out_type,
              scratch_types=_sup_scratch(d, with_cnt))
    if with_cnt:
        @functools.partial(pl.kernel, **kw)
        def k(table, src2d, dst2d, out, cnt_out, src_sup, dst_sup,
              rows0, rows1, acc_sh, gs0, gs1, ss0, ss1, ones_v, cnt_sh):
            body_fn(table, src2d, dst2d, out, cnt_out, src_sup, dst_sup,
                    rows0, rows1, acc_sh, gs0, gs1, ss0, ss1, ones_v, cnt_sh)
    else:
        @functools.partial(pl.kernel, **kw)
        def k(table, src2d, dst2d, out, src_sup, dst_sup,
              rows0, rows1, acc_sh, gs0, gs1, ss0, ss1):
            body_fn(table, src2d, dst2d, out, None, src_sup, dst_sup,
                    rows0, rows1, acc_sh, gs0, gs1, ss0, ss1, None, None)

    return k


@functools.lru_cache(maxsize=None)
def _aux1():
    d = 128
    dd = d // 16

    @functools.partial(
        pl.kernel,
        mesh=_sc_mesh(),
        out_type=[
            jax.ShapeDtypeStruct((2 * NP, d), F32),
            jax.ShapeDtypeStruct((2 * NP,), F32),
            jax.ShapeDtypeStruct((1024, d), F32),
            jax.ShapeDtypeStruct((1024, d), F32),
        ],
        scratch_types=_seg_scratch(d, True) + [
            pltpu.VMEM((32,), jnp.int32),
            pltpu.VMEM((32, d), F32),
            pltpu.SemaphoreType.DMA,
        ],
    )
    def k(gen_t, src8, dst8, x_t, idxg, e1_o, k_o, xm_o, gm_o,
          src0, src1, dst0, dst1, rows0, rows1, acc_sh, gs0, gs1,
          ones_v, cnt_sh, idx_v, grow_v, gsem2):
        c = lax.axis_index("c")
        s = lax.axis_index("s")
        w = s * 2 + c
        _gather32(x_t, idxg, xm_o, w, idx_v, grow_v, gsem2)
        _gather32(gen_t, idxg, gm_o, w, idx_v, grow_v, gsem2)
        _zero_rows(rows0, dd)
        base = s * RPS
        _zero_acc(rows0, base, acc_sh, cnt_sh)
        for j in range(CH // 16):
            ones_v[pl.ds(j * 16, 16)] = jnp.ones((16,), F32)
        plsc.subcore_barrier()
        _edge_loop(2, w, gen_t, src8, dst8, (src0, src1), (dst0, dst1),
                   (rows0, rows1), (gs0, gs1), ones_v, acc_sh, cnt_sh)
        plsc.subcore_barrier()
        ob = c * NP + base
        pltpu.sync_copy(acc_sh.at[pl.ds(base, RPS)], e1_o.at[pl.ds(ob, RPS)])
        pltpu.sync_copy(cnt_sh.at[pl.ds(base, RPS)],
                        k_o.at[pl.ds(ob, RPS)])

    return k


@functools.lru_cache(maxsize=None)
def _aux2():
    d = 128
    dd = d // 16

    @functools.partial(
        pl.kernel,
        mesh=_sc_mesh(),
        out_type=[
            jax.ShapeDtypeStruct((2 * NP, d), F32),
            jax.ShapeDtypeStruct((2 * NP, d), F32),
            jax.ShapeDtypeStruct((1024, d), F32),
            jax.ShapeDtypeStruct((1024, d), F32),
        ],
        scratch_types=_seg_scratch(d, False) + [
            pltpu.VMEM((32,), jnp.int32),
            pltpu.VMEM((32, d), F32),
            pltpu.SemaphoreType.DMA,
        ],
    )
    def k(lo_t, hi_t, ar8, dst8, h1lo_t, h1hi_t, idxg,
          e2lo_o, e2hi_o, mlo_o, mhi_o,
          src0, src1, dst0, dst1, rows0, rows1, acc_sh, gs0, gs1,
          idx_v, grow_v, gsem2):
        c = lax.axis_index("c")
        s = lax.axis_index("s")
        w = s * 2 + c
        _gather32(h1lo_t, idxg, mlo_o, w, idx_v, grow_v, gsem2)
        _gather32(h1hi_t, idxg, mhi_o, w, idx_v, grow_v, gsem2)
        base = s * RPS
        ob = c * NP + base
        srcs = (src0, src1)
        dsts = (dst0, dst1)
        rows = (rows0, rows1)
        gsem = (gs0, gs1)
        _zero_rows(rows0, dd)
        _zero_acc(rows0, base, acc_sh, None)
        plsc.subcore_barrier()
        _edge_loop(2, w, lo_t, ar8, dst8, srcs, dsts, rows, gsem, None,
                   acc_sh, None)
        plsc.subcore_barrier()
        pltpu.sync_copy(acc_sh.at[pl.ds(base, RPS)],
                        e2lo_o.at[pl.ds(ob, RPS)])
        _zero_rows(rows0, dd)
        _zero_acc(rows0, base, acc_sh, None)
        plsc.subcore_barrier()
        _edge_loop(2, w, hi_t, ar8, dst8, srcs, dsts, rows, gsem, None,
                   acc_sh, None)
        plsc.subcore_barrier()
        pltpu.sync_copy(acc_sh.at[pl.ds(base, RPS)],
                        e2hi_o.at[pl.ds(ob, RPS)])

    return k


def _trash(num):
    return 10000 + (jnp.arange(num, dtype=jnp.int32) % (NP - 10000))


def _spread_src(num, nt):
    return jnp.arange(num, dtype=jnp.int32) % nt


def _pad_edges(src, dst, e, nt):
    ep = -(-e // (32 * CH * SUP)) * (32 * CH * SUP)
    if ep == e:
        return src, dst, e
    pad = ep - e
    src_p = jnp.concatenate([src, _spread_src(pad, nt)])
    dst_p = jnp.concatenate([dst, _trash(pad)])
    return src_p, dst_p, ep


def _mm(a, w):
    return jnp.dot(a, w, preferred_element_type=F32)


def _gen_body(aggA, aggB, cntA, cntB, xb, wgl, bgl, wgr, wd1, bd1, wd2, bd2,
              gen_o):
    cnt = cntA[...] + cntB[...]
    mean0 = (aggA[...] + aggB[...]) / jnp.maximum(cnt, 1.0)
    h = jnp.maximum(_mm(mean0, wgl[...]) + bgl[...] + _mm(xb[...], wgr[...]),
                    0.0)
    t = jnp.maximum(_mm(h, wd1[...]) + bd1[...], 0.0)
    gen_o[...] = _mm(t, wd2[...]) + bd2[...]


def _conv1_body(aggA, aggB, e1A, e1B, cntA, cntB, kA, kB, xb, wl1, bl1, wr1,
                h1lo_o, h1hi_o, den_o):
    den = jnp.maximum(cntA[...] + cntB[...] + kA[...] + kB[...], 1.0)
    den_r = 1.0 / den
    mean1 = (aggA[...] + aggB[...] + e1A[...] + e1B[...]) * den_r
    h1 = jnp.maximum(_mm(mean1, wl1[...]) + bl1[...] + _mm(xb[...], wr1[...]),
                     0.0)
    h1lo_o[...] = h1[:, :128]
    h1hi_o[...] = h1[:, 128:]
    den_o[...] = den_r


def _new1_body(xm, gm, wl1, bl1, wr1, lo_o, hi_o):
    h1n = jnp.maximum(_mm(xm[...], wl1[...]) + bl1[...] +
                      _mm(gm[...], wr1[...]), 0.0)
    lo_o[...] = h1n[:, :128]
    hi_o[...] = h1n[:, 128:]


def _conv2_body(aloA, aloB, ahiA, ahiB, eloA, eloB, ehiA, ehiB, den, h1lo,
                h1hi, wl2, bl2, wr2, wp, bp, out_o):
    d = den[...]
    mlo = (aloA[...] + aloB[...] + eloA[...] + eloB[...]) * d
    mhi = (ahiA[...] + ahiB[...] + ehiA[...] + ehiB[...]) * d
    wl2v = wl2[...]
    wr2v = wr2[...]
    h2 = jnp.maximum(
        _mm(mlo, wl2v[:128]) + _mm(mhi, wl2v[128:]) + bl2[...] +
        _mm(h1lo[...], wr2v[:128]) + _mm(h1hi[...], wr2v[128:]), 0.0)
    out_o[...] = _mm(h2, wp[...]) + bp[...]


def _new2_body(h1mlo, h1mhi, h1nlo, h1nhi, wl2, bl2, wr2, wp, bp, out_o):
    wl2v = wl2[...]
    wr2v = wr2[...]
    h2n = jnp.maximum(
        _mm(h1mlo[...], wl2v[:128]) + _mm(h1mhi[...], wl2v[128:]) + bl2[...] +
        _mm(h1nlo[...], wr2v[:128]) + _mm(h1nhi[...], wr2v[128:]), 0.0)
    out_o[...] = _mm(h2n, wp[...]) + bp[...]


def _row_spec(w, two_part):
    nb = NP // BLK
    if two_part == 0:
        return pl.BlockSpec((BLK, w), lambda i: (i, 0))
    return pl.BlockSpec((BLK, w), lambda i, nb=nb: (i + nb, 0))


def _full_spec(shape):
    nd = len(shape)
    return pl.BlockSpec(shape, lambda i: (0,) * nd)


def kernel(x, edge_index, missing_indices, Wl1, bl1, Wr1, Wl2, bl2, Wr2,
           Wp, bp, Wgl, bgl, Wgr, Wd1, bd1, Wd2, bd2):
    n, dx = x.shape
    e = edge_index.shape[1]
    m = missing_indices.shape[0]
    src = edge_index[0].astype(jnp.int32)
    dst = edge_index[1].astype(jnp.int32)
    midx = missing_indices.astype(jnp.int32)
    mp = 1024
    x_pad = jnp.pad(x, ((0, NP - n), (0, 0)))
    src_p, dst_p, ep = _pad_edges(src, dst, e, n)
    src_m = jnp.concatenate([midx, jnp.zeros((mp - m,), jnp.int32)])
    src_m8 = jnp.concatenate([midx, _spread_src(8192 - m, n)])
    dst_m8 = jnp.concatenate([midx, _trash(8192 - m)])
    ar8 = jnp.concatenate([jnp.arange(mp, dtype=jnp.int32),
                           _spread_src(8192 - mp, mp)])

    bgl_r = bgl.reshape(1, -1)
    bd1_r = bd1.reshape(1, -1)
    bd2_r = bd2.reshape(1, -1)
    bl1_r = bl1.reshape(1, -1)
    bl2_r = bl2.reshape(1, -1)
    bp_r = bp.reshape(1, -1)

    src2 = src_p.reshape(-1, CH)
    dst2 = dst_p.reshape(-1, CH)
    agg, cnt = _segsum_sup(NP, 128, ep)(x_pad, src2, dst2)
    cnt2 = cnt.reshape(2 * NP, 1)

    nb = NP // BLK
    gen = pl.pallas_call(
        _gen_body,
        grid=(nb,),
        in_specs=[
            _row_spec(128, 0), _row_spec(128, 1),
            _row_spec(1, 0), _row_spec(1, 1),
            _row_spec(128, 0),
            _full_spec((128, 256)), _full_spec((1, 256)),
            _full_spec((128, 256)),
            _full_spec((256, 256)), _full_spec((1, 256)),
            _full_spec((256, 128)), _full_spec((1, 128)),
        ],
        out_specs=_row_spec(128, 0),
        out_shape=jax.ShapeDtypeStruct((NP, 128), F32),
    )(agg, agg, cnt2, cnt2, x_pad, Wgl, bgl_r, Wgr, Wd1, bd1_r, Wd2, bd2_r)

    e1, kcnt, xm, gm = _aux1()(gen, src_m8, dst_m8, x_pad, src_m)
    k2 = kcnt.reshape(2 * NP, 1)

    h1lo, h1hi, den_r = pl.pallas_call(
        _conv1_body,
        grid=(nb,),
        in_specs=[
            _row_spec(128, 0), _row_spec(128, 1),
            _row_spec(128, 0), _row_spec(128, 1),
            _row_spec(1, 0), _row_spec(1, 1),
            _row_spec(1, 0), _row_spec(1, 1),
            _row_spec(128, 0),
            _full_spec((128, 256)), _full_spec((1, 256)),
            _full_spec((128, 256)),
        ],
        out_specs=[_row_spec(128, 0), _row_spec(128, 0), _row_spec(1, 0)],
        out_shape=[
            jax.ShapeDtypeStruct((NP, 128), F32),
            jax.ShapeDtypeStruct((NP, 128), F32),
            jax.ShapeDtypeStruct((NP, 1), F32),
        ],
    )(agg, agg, e1, e1, cnt2, cnt2, k2, k2, x_pad, Wl1, bl1_r, Wr1)

    h1nlo, h1nhi = pl.pallas_call(
        _new1_body,
        grid=(mp // BLK,),
        in_specs=[
            _row_spec(128, 0), _row_spec(128, 0),
            _full_spec((128, 256)), _full_spec((1, 256)),
            _full_spec((128, 256)),
        ],
        out_specs=[_row_spec(128, 0), _row_spec(128, 0)],
        out_shape=[
            jax.ShapeDtypeStruct((mp, 128), F32),
            jax.ShapeDtypeStruct((mp, 128), F32),
        ],
    )(xm, gm, Wl1, bl1_r, Wr1)

    a2lo = _segsum_sup(NP, 128, ep, False)(h1lo, src2, dst2)
    a2hi = _segsum_sup(NP, 128, ep, False)(h1hi, src2, dst2)
    e2lo, e2hi, h1mlo, h1mhi = _aux2()(h1nlo, h1nhi, ar8, dst_m8,
                                       h1lo, h1hi, src_m)

    out_main = pl.pallas_call(
        _conv2_body,
        grid=(nb,),
        in_specs=[
            _row_spec(128, 0), _row_spec(128, 1),
            _row_spec(128, 0), _row_spec(128, 1),
            _row_spec(128, 0), _row_spec(128, 1),
            _row_spec(128, 0), _row_spec(128, 1),
            _row_spec(1, 0),
            _row_spec(128, 0), _row_spec(128, 0),
            _full_spec((256, 256)), _full_spec((1, 256)),
            _full_spec((256, 256)),
            _full_spec((256, 64)), _full_spec((1, 64)),
        ],
        out_specs=_row_spec(64, 0),
        out_shape=jax.ShapeDtypeStruct((NP, 64), F32),
    )(a2lo, a2lo, a2hi, a2hi, e2lo, e2lo, e2hi, e2hi, den_r, h1lo, h1hi,
      Wl2, bl2_r, Wr2, Wp, bp_r)

    out_new = pl.pallas_call(
        _new2_body,
        grid=(mp // BLK,),
        in_specs=[
            _row_spec(128, 0), _row_spec(128, 0),
            _row_spec(128, 0), _row_spec(128, 0),
            _full_spec((256, 256)), _full_spec((1, 256)),
            _full_spec((256, 256)),
            _full_spec((256, 64)), _full_spec((1, 64)),
        ],
        out_specs=_row_spec(64, 0),
        out_shape=jax.ShapeDtypeStruct((mp, 64), F32),
    )(h1mlo, h1mhi, h1nlo, h1nhi, Wl2, bl2_r, Wr2, Wp, bp_r)

    return jnp.concatenate([out_main[:n], out_new[:m]], axis=0)
